# baseline (speedup 1.0000x reference)
import functools

import jax
import jax.numpy as jnp
from jax import lax
from jax.experimental import pallas as pl
from jax.experimental.pallas import tpu as pltpu
from jax.experimental.pallas import tpu_sc as plsc

B = 16384
D_IN = 64
D_EMB = 2 * D_IN
DP = D_EMB + 8
NC = 2
NS = 16
NW = NC * NS
SUB = 128
NJ = B // (NW * SUB)
PW = NJ * SUB
L = 16


def _sc_body(table_hbm, lem_hbm, e1t_hbm, e2t_hbm, out_hbm,
             idx_v, rows_v, e1t_v, e2t_v, out_v, sem0, sem1):
    wid = lax.axis_index("s") * NC + lax.axis_index("c")
    base = wid * PW
    sems = (sem0, sem1)

    idx_copies = [
        pltpu.async_copy(lem_hbm.at[pl.ds(base + j * SUB, SUB)],
                         idx_v.at[j], sem0)
        for j in range(NJ)
    ]
    for c in idx_copies:
        c.wait()

    def start(j, b):
        r0 = base + j * SUB
        return (
            pltpu.async_copy(table_hbm.at[idx_v.at[j]],
                             rows_v.at[b].at[:, pl.ds(0, D_EMB)], sems[b]),
            pltpu.async_copy(e1t_hbm.at[:, pl.ds(r0, SUB)], e1t_v.at[b], sems[b]),
            pltpu.async_copy(e2t_hbm.at[:, pl.ds(r0, SUB)], e2t_v.at[b], sems[b]),
        )

    lane = lax.broadcasted_iota(jnp.int32, (L,), 0)

    def compute(j, b):
        def group(g, carry):
            r0g = g * L
            row_idx = lane + r0g

            def dot_half(eref, dbase):
                acc_a = plsc.load_gather(
                    rows_v.at[b],
                    [row_idx, jnp.full((L,), dbase, jnp.int32)]) \
                    * eref[b, 0, pl.ds(r0g, L)]
                acc_b = plsc.load_gather(
                    rows_v.at[b],
                    [row_idx, jnp.full((L,), dbase + 1, jnp.int32)]) \
                    * eref[b, 1, pl.ds(r0g, L)]
                for d in range(2, D_IN, 2):
                    acc_a += plsc.load_gather(
                        rows_v.at[b],
                        [row_idx, jnp.full((L,), dbase + d, jnp.int32)]) \
                        * eref[b, d, pl.ds(r0g, L)]
                    acc_b += plsc.load_gather(
                        rows_v.at[b],
                        [row_idx, jnp.full((L,), dbase + d + 1, jnp.int32)]) \
                        * eref[b, d + 1, pl.ds(r0g, L)]
                return acc_a + acc_b

            tot = dot_half(e1t_v, 0) + dot_half(e2t_v, D_IN)
            out_v[pl.ds(j * SUB + r0g, L)] = 1.0 / (1.0 + jnp.exp(-tot))
            return carry

        lax.fori_loop(0, SUB // L, group, 0)

    handles = start(0, 0)
    for j in range(NJ):
        b = j % 2
        if j + 1 < NJ:
            next_handles = start(j + 1, (j + 1) % 2)
        for h in handles:
            h.wait()
        compute(j, b)
        if j + 1 < NJ:
            handles = next_handles

    pltpu.sync_copy(out_v, out_hbm.at[pl.ds(base, PW)])


@jax.jit
def _run(table_pad, lemmas, e1t, e2t):
    mesh = plsc.VectorSubcoreMesh(core_axis_name="c", subcore_axis_name="s")
    f = functools.partial(
        pl.kernel,
        mesh=mesh,
        compiler_params=pltpu.CompilerParams(needs_layout_passes=False),
        out_type=jax.ShapeDtypeStruct((B,), jnp.float32),
        scratch_types=[
            pltpu.VMEM((NJ, SUB), jnp.int32),
            pltpu.VMEM((2, SUB, DP), jnp.float32),
            pltpu.VMEM((2, D_IN, SUB), jnp.float32),
            pltpu.VMEM((2, D_IN, SUB), jnp.float32),
            pltpu.VMEM((PW,), jnp.float32),
            pltpu.SemaphoreType.DMA,
            pltpu.SemaphoreType.DMA,
        ],
    )(_sc_body)
    return f(table_pad, lemmas, e1t, e2t)


def kernel(emb1, emb2, lemmas, lemma_embs):
    return _run(lemma_embs, lemmas, emb1.T, emb2.T)


# speedup vs baseline: 1.6930x; 1.6930x over previous
import functools

import jax
import jax.numpy as jnp
from jax import lax
from jax.experimental import pallas as pl
from jax.experimental.pallas import tpu as pltpu
from jax.experimental.pallas import tpu_sc as plsc

B = 16384
D_IN = 64
D_EMB = 2 * D_IN
NC = 2
NS = 16
NW = NC * NS
SUB = 128
NJ = B // (NW * SUB)
PW = NJ * SUB
L = 16
PP = L + 1


def _sc_body(table_hbm, lem_hbm, e1_hbm, e2_hbm, out_hbm,
             idx_v, rows_v, e1_v, e2_v, p_scr, out_v, sem0, sem1):
    wid = lax.axis_index("s") * NC + lax.axis_index("c")
    base = wid * PW
    sems = (sem0, sem1)

    idx_copies = [
        pltpu.async_copy(lem_hbm.at[pl.ds(base + j * SUB, SUB)],
                         idx_v.at[j], sem0)
        for j in range(NJ)
    ]
    for c in idx_copies:
        c.wait()

    def start(j, b):
        r0 = base + j * SUB
        return (
            pltpu.async_copy(table_hbm.at[idx_v.at[j]], rows_v.at[b], sems[b]),
            pltpu.async_copy(e1_hbm.at[pl.ds(r0, SUB)], e1_v.at[b], sems[b]),
            pltpu.async_copy(e2_hbm.at[pl.ds(r0, SUB)], e2_v.at[b], sems[b]),
        )

    lane = lax.broadcasted_iota(jnp.int32, (L,), 0)

    def compute(j, b):
        def group(g, carry):
            gbase = g * L
            for jj in range(L):
                r = gbase + jj
                acc = rows_v[b, r, pl.ds(0, L)] * e1_v[b, r, pl.ds(0, L)]
                for k in range(1, 4):
                    acc += rows_v[b, r, pl.ds(k * L, L)] * e1_v[b, r, pl.ds(k * L, L)]
                for k in range(4):
                    acc += (rows_v[b, r, pl.ds(D_IN + k * L, L)]
                            * e2_v[b, r, pl.ds(k * L, L)])
                p_scr[jj, pl.ds(0, L)] = acc
            tot = plsc.load_gather(p_scr, [lane, jnp.zeros((L,), jnp.int32)])
            for d in range(1, L):
                tot += plsc.load_gather(
                    p_scr, [lane, jnp.full((L,), d, jnp.int32)])
            out_v[pl.ds(j * SUB + gbase, L)] = 1.0 / (1.0 + jnp.exp(-tot))
            return carry

        lax.fori_loop(0, SUB // L, group, 0)

    handles = start(0, 0)
    for j in range(NJ):
        b = j % 2
        if j + 1 < NJ:
            next_handles = start(j + 1, (j + 1) % 2)
        for h in handles:
            h.wait()
        compute(j, b)
        if j + 1 < NJ:
            handles = next_handles

    pltpu.sync_copy(out_v, out_hbm.at[pl.ds(base, PW)])


@jax.jit
def _run(lemma_embs, lemmas, emb1, emb2):
    mesh = plsc.VectorSubcoreMesh(core_axis_name="c", subcore_axis_name="s")
    f = functools.partial(
        pl.kernel,
        mesh=mesh,
        compiler_params=pltpu.CompilerParams(needs_layout_passes=False),
        out_type=jax.ShapeDtypeStruct((B,), jnp.float32),
        scratch_types=[
            pltpu.VMEM((NJ, SUB), jnp.int32),
            pltpu.VMEM((2, SUB, D_EMB), jnp.float32),
            pltpu.VMEM((2, SUB, D_IN), jnp.float32),
            pltpu.VMEM((2, SUB, D_IN), jnp.float32),
            pltpu.VMEM((L, PP), jnp.float32),
            pltpu.VMEM((PW,), jnp.float32),
            pltpu.SemaphoreType.DMA,
            pltpu.SemaphoreType.DMA,
        ],
    )(_sc_body)
    return f(lemma_embs, lemmas, emb1, emb2)


def kernel(emb1, emb2, lemmas, lemma_embs):
    return _run(lemma_embs, lemmas, emb1, emb2)
